# baseline (speedup 1.0000x reference)
import functools

import jax
import jax.numpy as jnp
from jax import lax
from jax.experimental import pallas as pl
from jax.experimental.pallas import tpu as pltpu
from jax.experimental.pallas import tpu_sc as plsc

EMBED_D = 128
TABLE_ROWS = 20
N_ROWS = 62000
TC_ROWS = 38000
NUM_CORES = 2
NUM_SUBCORES = 16
NUM_WORKERS = NUM_CORES * NUM_SUBCORES
CHUNK = 400
NUM_CHUNKS = N_ROWS // CHUNK
MAX_ITERS = -(-NUM_CHUNKS // NUM_WORKERS)

_mesh = plsc.VectorSubcoreMesh(
    core_axis_name="c", subcore_axis_name="s",
    num_cores=NUM_CORES, num_subcores=NUM_SUBCORES)


@functools.partial(
    pl.kernel,
    mesh=_mesh,
    out_type=jax.ShapeDtypeStruct((N_ROWS, EMBED_D), jnp.float32),
    scratch_types=(
        [pltpu.VMEM_SHARED((TABLE_ROWS, EMBED_D), jnp.float32),
         pltpu.VMEM((2, CHUNK, EMBED_D), jnp.float32)]
        + [pltpu.VMEM((CHUNK,), jnp.int32) for _ in range(MAX_ITERS)]
        + [pltpu.SemaphoreType.DMA,
           pltpu.SemaphoreType.DMA,
           pltpu.SemaphoreType.DMA,
           pltpu.SemaphoreType.DMA]
    ),
)
def _embed_sc(idx_hbm, table_hbm, out_hbm, *scratch):
    table_v, rows_v = scratch[0], scratch[1]
    idx_v = scratch[2:2 + MAX_ITERS]
    sem_g, sem_s0, sem_s1, sem_i = scratch[2 + MAX_ITERS:]
    sem_s = (sem_s0, sem_s1)
    wid = lax.axis_index("s") * NUM_CORES + lax.axis_index("c")

    def chunk_id(k):
        return wid + k * NUM_WORKERS

    def out_slice(k):
        return out_hbm.at[pl.ds(chunk_id(k) * CHUNK, CHUNK)]

    @pl.when(lax.axis_index("s") == 0)
    def _():
        pltpu.sync_copy(table_hbm, table_v)

    plsc.subcore_barrier()

    for k in range(MAX_ITERS):

        @pl.when(chunk_id(k) < NUM_CHUNKS)
        def _():
            pltpu.async_copy(idx_hbm.at[pl.ds(chunk_id(k) * CHUNK, CHUNK)],
                             idx_v[k], sem_i)

    for k in range(MAX_ITERS):

        @pl.when(chunk_id(k) < NUM_CHUNKS)
        def _():
            pltpu.make_async_copy(
                idx_hbm.at[pl.ds(chunk_id(k) * CHUNK, CHUNK)],
                idx_v[k], sem_i).wait()

    for k in range(MAX_ITERS):
        buf = k % 2

        @pl.when(chunk_id(k) < NUM_CHUNKS)
        def _():
            if k >= 2:
                pltpu.make_async_copy(rows_v.at[buf], out_slice(k - 2),
                                      sem_s[buf]).wait()
            pltpu.async_copy(table_v.at[idx_v[k]], rows_v.at[buf],
                             sem_g).wait()
            pltpu.async_copy(rows_v.at[buf], out_slice(k), sem_s[buf])

    for k in range(max(MAX_ITERS - 2, 0), MAX_ITERS):
        buf = k % 2

        @pl.when(chunk_id(k) < NUM_CHUNKS)
        def _():
            pltpu.make_async_copy(rows_v.at[buf], out_slice(k),
                                  sem_s[buf]).wait()


TPAD = 32
BLK = 1000
NB_TC = TC_ROWS // BLK


def _tc_body(idx_ref, tab_ref, out_ref):
    idx = idx_ref[0, 0, :]
    onehot = (idx[:, None] == lax.broadcasted_iota(jnp.int32, (BLK, TPAD), 1)
              ).astype(jnp.float32)
    out_ref[...] = jnp.dot(onehot, tab_ref[...],
                           preferred_element_type=jnp.float32)


def _tc_embed(idx3, tab_p):
    return pl.pallas_call(
        _tc_body,
        grid=(NB_TC,),
        in_specs=[
            pl.BlockSpec((1, 1, BLK), lambda i: (i, 0, 0)),
            pl.BlockSpec((TPAD, EMBED_D), lambda i: (0, 0)),
        ],
        out_specs=pl.BlockSpec((BLK, EMBED_D), lambda i: (i, 0)),
        out_shape=jax.ShapeDtypeStruct((TC_ROWS, EMBED_D), jnp.float32),
    )(idx3, tab_p)


def kernel(atom_type_index, embedding_table):
    idx = atom_type_index.astype(jnp.int32)
    sc_out = _embed_sc(idx[:N_ROWS], embedding_table)
    idx3 = idx[N_ROWS:].reshape(NB_TC, 1, BLK)
    tab_p = jnp.zeros((TPAD, EMBED_D), jnp.float32).at[:20].set(embedding_table)
    tc_out = _tc_embed(idx3, tab_p)
    return jnp.concatenate([sc_out, tc_out], axis=0)


# speedup vs baseline: 1.8820x; 1.8820x over previous
import functools

import jax
import jax.numpy as jnp
from jax import lax
from jax.experimental import pallas as pl
from jax.experimental.pallas import tpu as pltpu
from jax.experimental.pallas import tpu_sc as plsc

EMBED_D = 128
TABLE_ROWS = 20
N_ROWS = 100000
NUM_CORES = 2
NUM_SUBCORES = 16
NUM_WORKERS = NUM_CORES * NUM_SUBCORES
CHUNK = 200
NUM_CHUNKS = N_ROWS // CHUNK
MAX_ITERS = -(-NUM_CHUNKS // NUM_WORKERS)
NBUF = 4

_mesh = plsc.VectorSubcoreMesh(
    core_axis_name="c", subcore_axis_name="s",
    num_cores=NUM_CORES, num_subcores=NUM_SUBCORES)


@functools.partial(
    pl.kernel,
    mesh=_mesh,
    out_type=jax.ShapeDtypeStruct((N_ROWS, EMBED_D), jnp.float32),
    scratch_types=(
        [pltpu.VMEM_SHARED((TABLE_ROWS, EMBED_D), jnp.float32)]
        + [pltpu.VMEM((CHUNK, EMBED_D), jnp.float32) for _ in range(NBUF)]
        + [pltpu.VMEM((CHUNK,), jnp.int32) for _ in range(MAX_ITERS)]
        + [pltpu.SemaphoreType.DMA for _ in range(2)]
        + [pltpu.SemaphoreType.DMA for _ in range(NBUF)]
        + [pltpu.SemaphoreType.DMA]
    ),
)
def _embed_sc(idx_hbm, table_hbm, out_hbm, *scratch):
    table_v = scratch[0]
    rows = scratch[1:1 + NBUF]
    idx_v = scratch[1 + NBUF:1 + NBUF + MAX_ITERS]
    sem_g = scratch[1 + NBUF + MAX_ITERS:3 + NBUF + MAX_ITERS]
    sem_s = scratch[3 + NBUF + MAX_ITERS:3 + 2 * NBUF + MAX_ITERS]
    sem_i = scratch[3 + 2 * NBUF + MAX_ITERS]
    wid = lax.axis_index("s") * NUM_CORES + lax.axis_index("c")

    def chunk_id(k):
        return wid + k * NUM_WORKERS

    def valid(k):
        return chunk_id(k) < NUM_CHUNKS

    def idx_slice(k):
        return idx_hbm.at[pl.ds(chunk_id(k) * CHUNK, CHUNK)]

    def out_slice(k):
        return out_hbm.at[pl.ds(chunk_id(k) * CHUNK, CHUNK)]

    def start_gather(k):
        _ = pltpu.async_copy(table_v.at[idx_v[k]], rows[k % NBUF], sem_g[k % 2])

    def wait_gather(k):
        pltpu.make_async_copy(table_v.at[idx_v[k]], rows[k % NBUF],
                              sem_g[k % 2]).wait()

    def start_write(k):
        _ = pltpu.async_copy(rows[k % NBUF], out_slice(k), sem_s[k % NBUF])

    def wait_write(k):
        pltpu.make_async_copy(rows[k % NBUF], out_slice(k),
                              sem_s[k % NBUF]).wait()

    def when_valid(k, fn):
        if k < MAX_ITERS - 1:
            fn()
        else:
            pl.when(valid(k))(fn)

    for k in range(MAX_ITERS):
        def _prefetch(k=k):
            pltpu.async_copy(idx_slice(k), idx_v[k], sem_i)

        when_valid(k, _prefetch)

    @pl.when(lax.axis_index("s") == 0)
    def _():
        pltpu.sync_copy(table_hbm, table_v)

    plsc.subcore_barrier()

    for k in range(MAX_ITERS):
        def _wait_prefetch(k=k):
            pltpu.make_async_copy(idx_slice(k), idx_v[k], sem_i).wait()

        when_valid(k, _wait_prefetch)

    start_gather(0)
    start_gather(1)

    for k in range(MAX_ITERS):
        def step(k=k):
            wait_gather(k)
            start_write(k)

        when_valid(k, step)

        if k + 2 < MAX_ITERS:
            def ahead(k=k):
                if k >= 2:
                    wait_write(k - 2)
                start_gather(k + 2)

            when_valid(k + 2, ahead)

    for j in range(MAX_ITERS):
        if j + 4 >= MAX_ITERS:
            when_valid(j, lambda j=j: wait_write(j))
        elif j + 4 == MAX_ITERS - 1:

            @pl.when(jnp.logical_not(valid(j + 4)))
            def _(j=j):
                wait_write(j)


def kernel(atom_type_index, embedding_table):
    idx = atom_type_index.astype(jnp.int32)
    return _embed_sc(idx, embedding_table)
